# baseline (speedup 1.0000x reference)
import functools

import jax
import jax.numpy as jnp
from jax import lax
from jax.experimental import pallas as pl
from jax.experimental.pallas import tpu as pltpu
from jax.experimental.pallas import tpu_sc as plsc

_NC = 2
_NS = 16
_NW = _NC * _NS
_CHUNK = 128
_D = 64


def _gather_body(n_chunks, b_per_w,
                 uid_hbm, iid_hbm, ut_hbm, it_hbm,
                 uout_hbm, iout_hbm,
                 uidx_v, iidx_v, urows_v, irows_v, sem):
    wid = lax.axis_index("s") * _NC + lax.axis_index("c")
    row0 = wid * n_chunks
    pltpu.sync_copy(uid_hbm.at[pl.ds(row0, n_chunks)], uidx_v)
    pltpu.sync_copy(iid_hbm.at[pl.ds(row0, n_chunks)], iidx_v)
    copies = []
    for j in range(n_chunks):
        copies.append(pltpu.async_copy(
            ut_hbm.at[uidx_v.at[j]], urows_v.at[pl.ds(j * _CHUNK, _CHUNK)], sem))
        copies.append(pltpu.async_copy(
            it_hbm.at[iidx_v.at[j]], irows_v.at[pl.ds(j * _CHUNK, _CHUNK)], sem))
    for c in copies:
        c.wait()
    base = wid * b_per_w
    pltpu.sync_copy(urows_v, uout_hbm.at[pl.ds(base, b_per_w)])
    pltpu.sync_copy(irows_v, iout_hbm.at[pl.ds(base, b_per_w)])


def _sc_gather(uid2, iid2, user_table, item_table):
    b = uid2.shape[0] * uid2.shape[1]
    b_per_w = b // _NW
    n_chunks = b_per_w // _CHUNK
    mesh = plsc.VectorSubcoreMesh(core_axis_name="c", subcore_axis_name="s")
    k = pl.kernel(
        functools.partial(_gather_body, n_chunks, b_per_w),
        mesh=mesh,
        compiler_params=pltpu.CompilerParams(use_tc_tiling_on_sc=False),
        out_type=[
            jax.ShapeDtypeStruct((b, _D), jnp.float32),
            jax.ShapeDtypeStruct((b, _D), jnp.float32),
        ],
        scratch_types=[
            pltpu.VMEM((n_chunks, _CHUNK), jnp.int32),
            pltpu.VMEM((n_chunks, _CHUNK), jnp.int32),
            pltpu.VMEM((b_per_w, _D), jnp.float32),
            pltpu.VMEM((b_per_w, _D), jnp.float32),
            pltpu.SemaphoreType.DMA,
        ],
    )
    return k(uid2, iid2, user_table, item_table)


def _mlp_body(u_ref, i_ref, w1a_ref, w1b_ref, b1_ref, w2_ref, b2_ref,
              w3_ref, b3_ref, o_ref):
    h = jnp.dot(u_ref[...], w1a_ref[...], preferred_element_type=jnp.float32)
    h = h + jnp.dot(i_ref[...], w1b_ref[...], preferred_element_type=jnp.float32)
    h = jnp.maximum(h + b1_ref[...], 0.0)
    h = jnp.dot(h, w2_ref[...], preferred_element_type=jnp.float32) + b2_ref[...]
    h = jnp.maximum(h, 0.0)
    o_ref[...] = jnp.dot(h, w3_ref[...], preferred_element_type=jnp.float32) + b3_ref[...]


def _tc_mlp(u_emb, i_emb, W1a, W1b, b1, W2, b2, W3, b3):
    b = u_emb.shape[0]
    blk = 2048
    grid = (b // blk,)
    row_spec = pl.BlockSpec((blk, _D), lambda g: (g, 0))
    full = lambda shape: pl.BlockSpec(shape, lambda g: (0, 0))
    return pl.pallas_call(
        _mlp_body,
        grid=grid,
        in_specs=[
            row_spec, row_spec,
            full(W1a.shape), full(W1b.shape), full(b1.shape),
            full(W2.shape), full(b2.shape),
            full(W3.shape), full(b3.shape),
        ],
        out_specs=pl.BlockSpec((blk, 1), lambda g: (g, 0)),
        out_shape=jax.ShapeDtypeStruct((b, 1), jnp.float32),
    )(u_emb, i_emb, W1a, W1b, b1, W2, b2, W3, b3)


def kernel(user_id, item_id, user_table, item_table, W1, b1, W2, b2, W3, b3):
    b = user_id.shape[0]
    uid2 = user_id.astype(jnp.int32).reshape(b // _CHUNK, _CHUNK)
    iid2 = item_id.astype(jnp.int32).reshape(b // _CHUNK, _CHUNK)
    u_emb, i_emb = _sc_gather(uid2, iid2, user_table, item_table)
    return _tc_mlp(
        u_emb, i_emb,
        W1[:_D], W1[_D:], b1.reshape(1, -1),
        W2, b2.reshape(1, -1),
        W3, b3.reshape(1, -1),
    )
